# baseline (speedup 1.0000x reference)
import functools

import jax
import jax.numpy as jnp
from jax import lax
from jax.experimental import pallas as pl
from jax.experimental.pallas import tpu as pltpu
from jax.experimental.pallas import tpu_sc as plsc

N = 50000
E = 1600000
DIM = 16
TW = 32
NG = 64
K = 2
LATENT = 4

NC = 2
NS = 16
NW = NC * NS
EPW = E // NW
CH = 200
NCHUNK = EPW // CH
AW = 24
N_PAD = 50048
ROWS_PER_TILE = N_PAD // NS


BN1 = 5000


def _prep_body(x_ref, pos_ref, w1i_ref, w1j_ref, b1_ref, ta_ref, tb_ref):
    xb = x_ref[...]
    posb = pos_ref[...]
    pad = jnp.zeros((BN1, TW - DIM - 3), jnp.float32)
    a = jnp.dot(xb, w1i_ref[...], preferred_element_type=jnp.float32)
    a = a + b1_ref[...]
    b = jnp.dot(xb, w1j_ref[...], preferred_element_type=jnp.float32)
    ta_ref[...] = jnp.concatenate([a, posb, pad], axis=1)
    tb_ref[...] = jnp.concatenate([b, posb, pad], axis=1)


def _prep_tables(x, pos, W1i, W1j, b1):
    grid = (N // BN1,)
    return pl.pallas_call(
        _prep_body,
        grid=grid,
        in_specs=[
            pl.BlockSpec((BN1, DIM), lambda i: (i, 0)),
            pl.BlockSpec((BN1, 3), lambda i: (i, 0)),
            pl.BlockSpec((DIM, DIM), lambda i: (0, 0)),
            pl.BlockSpec((DIM, DIM), lambda i: (0, 0)),
            pl.BlockSpec((1, DIM), lambda i: (0, 0)),
        ],
        out_specs=[
            pl.BlockSpec((BN1, TW), lambda i: (i, 0)),
            pl.BlockSpec((BN1, TW), lambda i: (i, 0)),
        ],
        out_shape=[
            jax.ShapeDtypeStruct((N, TW), jnp.float32),
            jax.ShapeDtypeStruct((N, TW), jnp.float32),
        ],
    )(x, pos, W1i, W1j, b1)


def _edge_body(ta_h, tb_h, src_h, dst_h, w1d_h, out0_h, out1_h,
               sidx0, sidx1, didx0, didx1, ta_v0, ta_v1, tb_v0, tb_v1,
               msg_v, w1d_v, s_sh, sa0, sa1, sb0, sb1):
    c = lax.axis_index("c")
    s = lax.axis_index("s")
    wid = s * NC + c

    pltpu.sync_copy(w1d_h, w1d_v)
    w1dv = w1d_v[...]
    w1ds = [w1dv[d] for d in range(DIM)]

    lane = lax.broadcasted_iota(jnp.int32, (16,), 0)
    zero16 = jnp.zeros((16,), jnp.float32)
    onecol = jnp.where(lane == DIM - (AW - 16), 1.0, 0.0).astype(jnp.float32)

    def _zb(r, _):
        msg_v[r, pl.ds(0, 16)] = zero16
        msg_v[r, pl.ds(AW - 16, 16)] = zero16
        return 0
    lax.fori_loop(0, CH, _zb, 0)

    row0 = s * ROWS_PER_TILE
    nfull = ROWS_PER_TILE // CH
    rem = ROWS_PER_TILE - nfull * CH
    for j in range(nfull):
        pltpu.sync_copy(msg_v.at[pl.ds(0, CH)],
                        s_sh.at[pl.ds(row0 + j * CH, CH)])
    pltpu.sync_copy(msg_v.at[pl.ds(0, rem)],
                    s_sh.at[pl.ds(row0 + nfull * CH, rem)])

    def _ob(r, _):
        msg_v[r, pl.ds(AW - 16, 16)] = onecol
        return 0
    lax.fori_loop(0, CH, _ob, 0)

    plsc.subcore_barrier()

    magic = jnp.full((16,), 0x5F3759DF, jnp.int32)

    bufs = ((sidx0, didx0, ta_v0, tb_v0, sa0, sb0),
            (sidx1, didx1, ta_v1, tb_v1, sa1, sb1))

    def _prefetch(ci, b):
        sidx, didx, ta_v, tb_v, sa, sb = bufs[b]
        base = wid * EPW + ci * CH
        pltpu.sync_copy(src_h.at[pl.ds(base, CH)], sidx)
        pltpu.sync_copy(dst_h.at[pl.ds(base, CH)], didx)
        pltpu.make_async_copy(ta_h.at[didx], ta_v, sa).start()
        pltpu.make_async_copy(tb_h.at[sidx], tb_v, sb).start()

    def _consume(b):
        sidx, didx, ta_v, tb_v, sa, sb = bufs[b]
        pltpu.make_async_copy(ta_h.at[didx], ta_v, sa).wait()
        pltpu.make_async_copy(tb_h.at[sidx], tb_v, sb).wait()

        @plsc.parallel_loop(0, CH, step=1, unroll=16)
        def _edge(e):
            arow = ta_v[e, pl.ds(0, 16)]
            brow = tb_v[e, pl.ds(0, 16)]
            rel = ta_v[e, pl.ds(TW - 16, 16)] - tb_v[e, pl.ds(TW - 16, 16)]
            r2 = rel * rel
            o = DIM - (TW - 16)
            d2 = r2[o] + r2[o + 1] + r2[o + 2]
            d2v = jnp.broadcast_to(d2, (16,))
            ii = magic - (plsc.bitcast(d2v, jnp.int32) >> 1)
            y = plsc.bitcast(ii, jnp.float32)
            y = y * (1.5 - 0.5 * d2v * y * y)
            y = y * (1.5 - 0.5 * d2v * y * y)
            dist = d2v * y
            h1 = arow + brow + dist * w1dv
            sg = 1.0 / (1.0 + jnp.exp(-h1))
            msg_v[e, pl.ds(0, 16)] = h1 * sg

        pltpu.sync_copy(msg_v, s_sh.at[didx], add=True)

    _prefetch(0, 0)

    def _pair(p, _):
        ci = 2 * p
        _prefetch(ci + 1, 1)
        _consume(0)

        @pl.when(ci + 2 < NCHUNK)
        def _():
            _prefetch(ci + 2, 0)

        _consume(1)
        return 0

    lax.fori_loop(0, NCHUNK // 2, _pair, 0)

    plsc.subcore_barrier()

    def _dump(out_h):
        for j in range(nfull):
            pltpu.sync_copy(s_sh.at[pl.ds(row0 + j * CH, CH)],
                            msg_v.at[pl.ds(0, CH)])
            pltpu.sync_copy(msg_v.at[pl.ds(0, CH)],
                            out_h.at[pl.ds(row0 + j * CH, CH)])
        pltpu.sync_copy(s_sh.at[pl.ds(row0 + nfull * CH, rem)],
                        msg_v.at[pl.ds(0, rem)])
        pltpu.sync_copy(msg_v.at[pl.ds(0, rem)],
                        out_h.at[pl.ds(row0 + nfull * CH, rem)])

    @pl.when(c == 0)
    def _():
        _dump(out0_h)

    @pl.when(c == 1)
    def _():
        _dump(out1_h)


def _edge_stage(ta, tb, src, dst, w1d):
    mesh = plsc.VectorSubcoreMesh(core_axis_name="c", subcore_axis_name="s")
    f = functools.partial(
        pl.kernel,
        out_type=[jax.ShapeDtypeStruct((N_PAD, AW), jnp.float32),
                  jax.ShapeDtypeStruct((N_PAD, AW), jnp.float32)],
        mesh=mesh,
        compiler_params=pltpu.CompilerParams(use_tc_tiling_on_sc=False,
                                             needs_layout_passes=False),
        scratch_types=[
            pltpu.VMEM((CH,), jnp.int32),
            pltpu.VMEM((CH,), jnp.int32),
            pltpu.VMEM((CH,), jnp.int32),
            pltpu.VMEM((CH,), jnp.int32),
            pltpu.VMEM((CH, TW), jnp.float32),
            pltpu.VMEM((CH, TW), jnp.float32),
            pltpu.VMEM((CH, TW), jnp.float32),
            pltpu.VMEM((CH, TW), jnp.float32),
            pltpu.VMEM((CH, AW), jnp.float32),
            pltpu.VMEM((16,), jnp.float32),
            pltpu.VMEM_SHARED((N_PAD, AW), jnp.float32),
            pltpu.SemaphoreType.DMA,
            pltpu.SemaphoreType.DMA,
            pltpu.SemaphoreType.DMA,
            pltpu.SemaphoreType.DMA,
        ],
    )(_edge_body)
    return f(ta, tb, src, dst, w1d)


BN3 = 5000
SUB = 200
NB3 = N // BN3


def _fin_body(s0_ref, s1_ref, batch_ref, w2_ref, b2_ref, wp_ref, bp_ref,
              wz_ref, bzt_ref, s_ref, zs_ref):
    g = pl.program_id(0)
    t = s0_ref[...] + s1_ref[...]
    ssum = t[:, 0:DIM]
    cnt = t[:, DIM:DIM + 1]
    aggs = jnp.dot(ssum, w2_ref[...], preferred_element_type=jnp.float32)
    aggs = aggs + cnt * b2_ref[...]
    agg = aggs / jnp.maximum(cnt, 1.0)
    h = jnp.maximum(agg, 0.0)
    logits = jnp.dot(h, wp_ref[...], preferred_element_type=jnp.float32)
    logits = logits + bp_ref[...]
    mx = jnp.max(logits, axis=1, keepdims=True)
    ex = jnp.exp(logits - mx)
    sm = ex / jnp.sum(ex, axis=1, keepdims=True)
    s_ref[...] = sm

    hz = jnp.dot(h, wz_ref[...], preferred_element_type=jnp.float32)
    wv = jnp.concatenate([sm[:, 0:1] * hz, sm[:, 1:2] * hz], axis=1)
    bb = batch_ref[...]

    @pl.when(g == 0)
    def _():
        zs_ref[...] = jnp.zeros_like(zs_ref)

    acc = zs_ref[...]
    iot = lax.broadcasted_iota(jnp.int32, (SUB, NG), 1)
    for i in range(BN3 // SUB):
        wvs = wv[i * SUB:(i + 1) * SUB]
        bbs = bb[i * SUB:(i + 1) * SUB]
        oh = (iot == bbs).astype(jnp.float32)
        acc = acc + lax.dot_general(oh, wvs, (((0,), (0,)), ((), ())),
                                    preferred_element_type=jnp.float32)
    islast = jnp.where(g == NB3 - 1, 1.0, 0.0)
    zs_ref[...] = acc + islast * bzt_ref[...]


def _finalize(spart0, spart1, batch2d, W2, b2, Wp, bp, Wz, bzt):
    grid = (NB3,)
    return pl.pallas_call(
        _fin_body,
        grid=grid,
        in_specs=[
            pl.BlockSpec((BN3, AW), lambda i: (i, 0)),
            pl.BlockSpec((BN3, AW), lambda i: (i, 0)),
            pl.BlockSpec((BN3, 1), lambda i: (i, 0)),
            pl.BlockSpec((DIM, DIM), lambda i: (0, 0)),
            pl.BlockSpec((1, DIM), lambda i: (0, 0)),
            pl.BlockSpec((DIM, K), lambda i: (0, 0)),
            pl.BlockSpec((1, K), lambda i: (0, 0)),
            pl.BlockSpec((DIM, LATENT), lambda i: (0, 0)),
            pl.BlockSpec((1, K * LATENT), lambda i: (0, 0)),
        ],
        out_specs=[
            pl.BlockSpec((BN3, K), lambda i: (i, 0)),
            pl.BlockSpec((NG, K * LATENT), lambda i: (0, 0)),
        ],
        out_shape=[
            jax.ShapeDtypeStruct((N, K), jnp.float32),
            jax.ShapeDtypeStruct((NG, K * LATENT), jnp.float32),
        ],
    )(spart0, spart1, batch2d, W2, b2, Wp, bp, Wz, bzt)


def kernel(x, pos, edge_index, batch, W1, b1, W2, b2, Wp, bp, Wz, bz):
    src = edge_index[0].astype(jnp.int32)
    dst = edge_index[1].astype(jnp.int32)
    W1i = W1[0:DIM]
    W1j = W1[DIM:2 * DIM]
    w1d = W1[2 * DIM]
    b1r = b1.reshape(1, DIM)

    ta, tb = _prep_tables(x, pos, W1i, W1j, b1r)
    spart0, spart1 = _edge_stage(ta, tb, src, dst, w1d)

    batch2d = batch.astype(jnp.int32).reshape(N, 1)
    bzt = jnp.tile(bz, K).reshape(1, K * LATENT)
    s, zs = _finalize(spart0, spart1, batch2d, W2, b2.reshape(1, DIM),
                      Wp, bp.reshape(1, K), Wz, bzt)
    z = zs.reshape(NG, K, LATENT)
    return (z, s)


# speedup vs baseline: 1.3340x; 1.3340x over previous
import functools

import jax
import jax.numpy as jnp
from jax import lax
from jax.experimental import pallas as pl
from jax.experimental.pallas import tpu as pltpu
from jax.experimental.pallas import tpu_sc as plsc

N = 50000
E = 1600000
DIM = 16
TW = 24
NG = 64
K = 2
LATENT = 4

NC = 2
NS = 16
NW = NC * NS
EPW = E // NW
CH = 400
NCHUNK = EPW // CH
AW = 24
N_PAD = 50048
ROWS_PER_TILE = N_PAD // NS


BN1 = 5000


def _prep_body(x_ref, pos_ref, w1i_ref, w1j_ref, b1_ref, ta_ref, tb_ref):
    xb = x_ref[...]
    posb = pos_ref[...]
    pad = jnp.zeros((BN1, TW - DIM - 3), jnp.float32)
    a = jnp.dot(xb, w1i_ref[...], preferred_element_type=jnp.float32)
    a = a + b1_ref[...]
    b = jnp.dot(xb, w1j_ref[...], preferred_element_type=jnp.float32)
    ta_ref[...] = jnp.concatenate([a, posb, pad], axis=1)
    tb_ref[...] = jnp.concatenate([b, posb, pad], axis=1)


def _prep_tables(x, pos, W1i, W1j, b1):
    grid = (N // BN1,)
    return pl.pallas_call(
        _prep_body,
        grid=grid,
        in_specs=[
            pl.BlockSpec((BN1, DIM), lambda i: (i, 0)),
            pl.BlockSpec((BN1, 3), lambda i: (i, 0)),
            pl.BlockSpec((DIM, DIM), lambda i: (0, 0)),
            pl.BlockSpec((DIM, DIM), lambda i: (0, 0)),
            pl.BlockSpec((1, DIM), lambda i: (0, 0)),
        ],
        out_specs=[
            pl.BlockSpec((BN1, TW), lambda i: (i, 0)),
            pl.BlockSpec((BN1, TW), lambda i: (i, 0)),
        ],
        out_shape=[
            jax.ShapeDtypeStruct((N, TW), jnp.float32),
            jax.ShapeDtypeStruct((N, TW), jnp.float32),
        ],
    )(x, pos, W1i, W1j, b1)


def _edge_body(ta_h, tb_h, src_h, dst_h, w1d_h, out0_h, out1_h,
               sidx0, sidx1, didx0, didx1, ta_v0, ta_v1, tb_v0, tb_v1,
               msg_v, w1d_v, s_sh, sa0, sa1, sb0, sb1):
    c = lax.axis_index("c")
    s = lax.axis_index("s")
    wid = s * NC + c

    pltpu.sync_copy(w1d_h, w1d_v)
    w1dv = w1d_v[...]
    w1ds = [w1dv[d] for d in range(DIM)]

    lane = lax.broadcasted_iota(jnp.int32, (16,), 0)
    zero16 = jnp.zeros((16,), jnp.float32)
    onecol = jnp.where(lane == DIM - (AW - 16), 1.0, 0.0).astype(jnp.float32)

    def _zb(r, _):
        msg_v[r, pl.ds(0, 16)] = zero16
        msg_v[r, pl.ds(AW - 16, 16)] = zero16
        return 0
    lax.fori_loop(0, CH, _zb, 0)

    row0 = s * ROWS_PER_TILE
    nfull = ROWS_PER_TILE // CH
    rem = ROWS_PER_TILE - nfull * CH
    for j in range(nfull):
        pltpu.sync_copy(msg_v.at[pl.ds(0, CH)],
                        s_sh.at[pl.ds(row0 + j * CH, CH)])
    pltpu.sync_copy(msg_v.at[pl.ds(0, rem)],
                    s_sh.at[pl.ds(row0 + nfull * CH, rem)])

    def _ob(r, _):
        msg_v[r, pl.ds(AW - 16, 16)] = onecol
        return 0
    lax.fori_loop(0, CH, _ob, 0)

    plsc.subcore_barrier()

    magic = jnp.int32(0x5F3759DF)

    bufs = ((sidx0, didx0, ta_v0, tb_v0, sa0, sb0),
            (sidx1, didx1, ta_v1, tb_v1, sa1, sb1))

    def _prefetch(ci, b):
        sidx, didx, ta_v, tb_v, sa, sb = bufs[b]
        base = wid * EPW + ci * CH
        pltpu.sync_copy(src_h.at[pl.ds(base, CH)], sidx)
        pltpu.sync_copy(dst_h.at[pl.ds(base, CH)], didx)
        pltpu.make_async_copy(ta_h.at[didx], ta_v, sa).start()
        pltpu.make_async_copy(tb_h.at[sidx], tb_v, sb).start()

    def _consume(b):
        sidx, didx, ta_v, tb_v, sa, sb = bufs[b]
        pltpu.make_async_copy(ta_h.at[didx], ta_v, sa).wait()
        pltpu.make_async_copy(tb_h.at[sidx], tb_v, sb).wait()

        @plsc.parallel_loop(0, CH, step=1, unroll=8)
        def _edge(e):
            arow = ta_v[e, pl.ds(0, 16)]
            brow = tb_v[e, pl.ds(0, 16)]
            rel = ta_v[e, pl.ds(TW - 16, 16)] - tb_v[e, pl.ds(TW - 16, 16)]
            r2 = rel * rel
            o = DIM - (TW - 16)
            d2 = r2[o] + r2[o + 1] + r2[o + 2]
            ii = magic - (lax.bitcast_convert_type(d2, jnp.int32) >> 1)
            y = lax.bitcast_convert_type(ii, jnp.float32)
            y = y * (1.5 - 0.5 * d2 * y * y)
            dist = d2 * y
            h1 = arow + brow + dist * w1dv
            sg = 1.0 / (1.0 + jnp.exp(-h1))
            msg_v[e, pl.ds(0, 16)] = h1 * sg

        pltpu.sync_copy(msg_v, s_sh.at[didx], add=True)

    _prefetch(0, 0)

    def _pair(p, _):
        ci = 2 * p
        _prefetch(ci + 1, 1)
        _consume(0)

        @pl.when(ci + 2 < NCHUNK)
        def _():
            _prefetch(ci + 2, 0)

        _consume(1)
        return 0

    lax.fori_loop(0, NCHUNK // 2, _pair, 0)
    if NCHUNK % 2 == 1:
        _consume(0)

    plsc.subcore_barrier()

    def _dump(out_h):
        for j in range(nfull):
            pltpu.sync_copy(s_sh.at[pl.ds(row0 + j * CH, CH)],
                            msg_v.at[pl.ds(0, CH)])
            pltpu.sync_copy(msg_v.at[pl.ds(0, CH)],
                            out_h.at[pl.ds(row0 + j * CH, CH)])
        pltpu.sync_copy(s_sh.at[pl.ds(row0 + nfull * CH, rem)],
                        msg_v.at[pl.ds(0, rem)])
        pltpu.sync_copy(msg_v.at[pl.ds(0, rem)],
                        out_h.at[pl.ds(row0 + nfull * CH, rem)])

    @pl.when(c == 0)
    def _():
        _dump(out0_h)

    @pl.when(c == 1)
    def _():
        _dump(out1_h)


def _edge_stage(ta, tb, src, dst, w1d):
    mesh = plsc.VectorSubcoreMesh(core_axis_name="c", subcore_axis_name="s")
    f = functools.partial(
        pl.kernel,
        out_type=[jax.ShapeDtypeStruct((N_PAD, AW), jnp.float32),
                  jax.ShapeDtypeStruct((N_PAD, AW), jnp.float32)],
        mesh=mesh,
        compiler_params=pltpu.CompilerParams(use_tc_tiling_on_sc=False,
                                             needs_layout_passes=False),
        scratch_types=[
            pltpu.VMEM((CH,), jnp.int32),
            pltpu.VMEM((CH,), jnp.int32),
            pltpu.VMEM((CH,), jnp.int32),
            pltpu.VMEM((CH,), jnp.int32),
            pltpu.VMEM((CH, TW), jnp.float32),
            pltpu.VMEM((CH, TW), jnp.float32),
            pltpu.VMEM((CH, TW), jnp.float32),
            pltpu.VMEM((CH, TW), jnp.float32),
            pltpu.VMEM((CH, AW), jnp.float32),
            pltpu.VMEM((16,), jnp.float32),
            pltpu.VMEM_SHARED((N_PAD, AW), jnp.float32),
            pltpu.SemaphoreType.DMA,
            pltpu.SemaphoreType.DMA,
            pltpu.SemaphoreType.DMA,
            pltpu.SemaphoreType.DMA,
        ],
    )(_edge_body)
    return f(ta, tb, src, dst, w1d)


BN3 = 5000
SUB = 200
NB3 = N // BN3


def _fin_body(s0_ref, s1_ref, batch_ref, w2_ref, b2_ref, wp_ref, bp_ref,
              wz_ref, bzt_ref, s_ref, zs_ref):
    g = pl.program_id(0)
    t = s0_ref[...] + s1_ref[...]
    ssum = t[:, 0:DIM]
    cnt = t[:, DIM:DIM + 1]
    aggs = jnp.dot(ssum, w2_ref[...], preferred_element_type=jnp.float32)
    aggs = aggs + cnt * b2_ref[...]
    agg = aggs / jnp.maximum(cnt, 1.0)
    h = jnp.maximum(agg, 0.0)
    logits = jnp.dot(h, wp_ref[...], preferred_element_type=jnp.float32)
    logits = logits + bp_ref[...]
    mx = jnp.max(logits, axis=1, keepdims=True)
    ex = jnp.exp(logits - mx)
    sm = ex / jnp.sum(ex, axis=1, keepdims=True)
    s_ref[...] = sm

    hz = jnp.dot(h, wz_ref[...], preferred_element_type=jnp.float32)
    wv = jnp.concatenate([sm[:, 0:1] * hz, sm[:, 1:2] * hz], axis=1)
    bb = batch_ref[...]

    @pl.when(g == 0)
    def _():
        zs_ref[...] = jnp.zeros_like(zs_ref)

    acc = zs_ref[...]
    iot = lax.broadcasted_iota(jnp.int32, (SUB, NG), 1)
    for i in range(BN3 // SUB):
        wvs = wv[i * SUB:(i + 1) * SUB]
        bbs = bb[i * SUB:(i + 1) * SUB]
        oh = (iot == bbs).astype(jnp.float32)
        acc = acc + lax.dot_general(oh, wvs, (((0,), (0,)), ((), ())),
                                    preferred_element_type=jnp.float32)
    islast = jnp.where(g == NB3 - 1, 1.0, 0.0)
    zs_ref[...] = acc + islast * bzt_ref[...]


def _finalize(spart0, spart1, batch2d, W2, b2, Wp, bp, Wz, bzt):
    grid = (NB3,)
    return pl.pallas_call(
        _fin_body,
        grid=grid,
        in_specs=[
            pl.BlockSpec((BN3, AW), lambda i: (i, 0)),
            pl.BlockSpec((BN3, AW), lambda i: (i, 0)),
            pl.BlockSpec((BN3, 1), lambda i: (i, 0)),
            pl.BlockSpec((DIM, DIM), lambda i: (0, 0)),
            pl.BlockSpec((1, DIM), lambda i: (0, 0)),
            pl.BlockSpec((DIM, K), lambda i: (0, 0)),
            pl.BlockSpec((1, K), lambda i: (0, 0)),
            pl.BlockSpec((DIM, LATENT), lambda i: (0, 0)),
            pl.BlockSpec((1, K * LATENT), lambda i: (0, 0)),
        ],
        out_specs=[
            pl.BlockSpec((BN3, K), lambda i: (i, 0)),
            pl.BlockSpec((NG, K * LATENT), lambda i: (0, 0)),
        ],
        out_shape=[
            jax.ShapeDtypeStruct((N, K), jnp.float32),
            jax.ShapeDtypeStruct((NG, K * LATENT), jnp.float32),
        ],
    )(spart0, spart1, batch2d, W2, b2, Wp, bp, Wz, bzt)


def kernel(x, pos, edge_index, batch, W1, b1, W2, b2, Wp, bp, Wz, bz):
    src = edge_index[0].astype(jnp.int32)
    dst = edge_index[1].astype(jnp.int32)
    W1i = W1[0:DIM]
    W1j = W1[DIM:2 * DIM]
    w1d = W1[2 * DIM]
    b1r = b1.reshape(1, DIM)

    ta, tb = _prep_tables(x, pos, W1i, W1j, b1r)
    spart0, spart1 = _edge_stage(ta, tb, src, dst, w1d)

    batch2d = batch.astype(jnp.int32).reshape(N, 1)
    bzt = jnp.tile(bz, K).reshape(1, K * LATENT)
    s, zs = _finalize(spart0, spart1, batch2d, W2, b2.reshape(1, DIM),
                      Wp, bp.reshape(1, K), Wz, bzt)
    z = zs.reshape(NG, K, LATENT)
    return (z, s)
